# baseline (speedup 1.0000x reference)
import functools

import jax
import jax.numpy as jnp
from jax import lax
from jax.experimental import pallas as pl
from jax.experimental.pallas import tpu as pltpu
from jax.experimental.pallas import tpu_sc as plsc

VOCAB = 1000000
EMBED_DIM = 64
BATCH = 16384
N_FIELDS = 26

NC, NS = 2, 16
NW = NC * NS
B = BATCH * N_FIELDS
B_PER_W = B // NW
IDX_W = 128
NG = B_PER_W // IDX_W
NBUF = 8
NROUND = NG // NBUF

NFULL_VB = VOCAB // 128
TAIL_V = VOCAB - NFULL_VB * 128
T5_BLKS = VOCAB * EMBED_DIM // 1024

_mesh = plsc.VectorSubcoreMesh(core_axis_name="c", subcore_axis_name="s")

def _transpose_block(src, dst, nt):
    iota = lax.iota(jnp.int32, 16)
    for t in range(nt):
        for s in range(8):
            vbase = (t * 8 + s) * 2
            for grp in range(8):
                e0 = (grp * 16) % 64
                col = vbase + (grp * 16) // 64
                vec = plsc.load_gather(
                    src, [iota + e0, jnp.full((16,), col, jnp.int32)])
                dst[t, s, pl.ds(grp * 16, 16)] = vec


@functools.partial(
    pl.kernel,
    out_type=jax.ShapeDtypeStruct((T5_BLKS, 8, 128), jnp.float32),
    mesh=_mesh,
    scratch_types=[
        pltpu.VMEM((64, 128), jnp.float32),
        pltpu.VMEM((64, 128), jnp.float32),
        pltpu.VMEM((8, 8, 128), jnp.float32),
        pltpu.VMEM((8, 8, 128), jnp.float32),
        pltpu.SemaphoreType.DMA,
        pltpu.SemaphoreType.DMA,
        pltpu.SemaphoreType.DMA,
        pltpu.SemaphoreType.DMA,
    ],
    compiler_params=pltpu.CompilerParams(use_tc_tiling_on_sc=True,
                                         needs_layout_passes=False),
)
def _sc_detile(wt, tail128, t5, in_a, in_b, out_a, out_b, ia, ib, oa, ob):
    wid = lax.axis_index("s") * NC + lax.axis_index("c")
    ins, outs, isems, osems = (in_a, in_b), (out_a, out_b), (ia, ib), (oa, ob)
    nfull = jnp.where(wid < 4, 245, 244)

    def fire_in(i, b):
        vb = wid + NW * i
        pltpu.async_copy(wt.at[:, pl.ds(vb * 128, 128)], ins[b], isems[b])

    def wait_in(b):
        pltpu.make_async_copy(wt.at[:, pl.ds(0, 128)], ins[b], isems[b]).wait()

    def wait_out(b):
        pltpu.make_async_copy(t5.at[pl.ds(0, 8)], outs[b], osems[b]).wait()

    for b in range(2):
        fire_in(b, b)

    def step(i):
        for b in range(2):
            @pl.when((i * 2 + b) < nfull)
            def _():
                j = i * 2 + b
                vb = wid + NW * j
                wait_in(b)
                @pl.when(j >= 2)
                def _():
                    wait_out(b)
                _transpose_block(ins[b], outs[b], 8)
                pltpu.async_copy(outs[b], t5.at[pl.ds(vb * 8, 8)], osems[b])
                @pl.when((j + 2) < nfull)
                def _():
                    fire_in(j + 2, b)

    pl.loop(0, 123)(step)
    for b in range(2):
        wait_out(b)

    @pl.when(wid == 4)
    def _():
        pltpu.sync_copy(tail128, in_a)
        _transpose_block(in_a, out_a, 4)
        pltpu.sync_copy(out_a.at[pl.ds(0, 4)],
                        t5.at[pl.ds(NFULL_VB * 8, 4)])


_scratch = (
    [pltpu.VMEM((NG, IDX_W), jnp.int32)]
    + [pltpu.VMEM((IDX_W, EMBED_DIM), jnp.float32) for _ in range(NBUF)]
    + [pltpu.SemaphoreType.DMA for _ in range(2 * NBUF)]
)


@functools.partial(
    pl.kernel,
    out_type=jax.ShapeDtypeStruct((B, EMBED_DIM), jnp.float32),
    mesh=_mesh,
    scratch_types=_scratch,
    compiler_params=pltpu.CompilerParams(use_tc_tiling_on_sc=False),
)
def _sc_gather(table, idx, out, idx_v, *bufs_and_sems):
    bufs = bufs_and_sems[:NBUF]
    gsems = bufs_and_sems[NBUF:2 * NBUF]
    ssems = bufs_and_sems[2 * NBUF:]
    wid = lax.axis_index("s") * NC + lax.axis_index("c")
    base = wid * B_PER_W
    pltpu.sync_copy(idx.at[wid], idx_v)

    def fire(g, b):
        pltpu.async_copy(table.at[idx_v.at[g]], bufs[b], gsems[b])

    def drain(b, sem):
        pltpu.make_async_copy(table.at[pl.ds(0, IDX_W)], bufs[b], sem).wait()

    for b in range(NBUF):
        fire(b, b)

    def step(i):
        for b in range(NBUF):
            g = i * NBUF + b
            drain(b, gsems[b])
            pltpu.async_copy(bufs[b], out.at[pl.ds(base + g * IDX_W, IDX_W)],
                             ssems[b])
        for b in range(NBUF):
            g_next = (i + 1) * NBUF + b

            @pl.when(g_next < NG)
            def _():
                drain(b, ssems[b])
                fire(g_next, b)

    pl.loop(0, NROUND)(step)
    for b in range(NBUF):
        drain(b, ssems[b])


def kernel(context, W):
    wt = W.T
    tail128 = jnp.pad(wt[:, NFULL_VB * 128:], ((0, 0), (0, 128 - TAIL_V)))
    t5 = _sc_detile(wt, tail128)
    table = t5.reshape(VOCAB, EMBED_DIM)
    idx = context.astype(jnp.int32).reshape(NW, NG, IDX_W)
    out = _sc_gather(table, idx)
    return out.reshape(BATCH, N_FIELDS, EMBED_DIM)


# speedup vs baseline: 1.2797x; 1.2797x over previous
import functools

import jax
import jax.numpy as jnp
from jax import lax
from jax.experimental import pallas as pl
from jax.experimental.pallas import tpu as pltpu
from jax.experimental.pallas import tpu_sc as plsc

VOCAB = 1000000
EMBED_DIM = 64
BATCH = 16384
N_FIELDS = 26

NC, NS = 2, 16
NW = NC * NS
B = BATCH * N_FIELDS
B_PER_W = B // NW
IDX_W = 128
NG = B_PER_W // IDX_W
NBUF = 8
NROUND = NG // NBUF

NFULL_VB = VOCAB // 128
TAIL_V = VOCAB - NFULL_VB * 128
T5_BLKS = VOCAB * EMBED_DIM // 1024

_mesh = plsc.VectorSubcoreMesh(core_axis_name="c", subcore_axis_name="s")

def _transpose_block(src, dst, nt):
    iota = lax.iota(jnp.int32, 16)
    work = []
    for t in range(nt):
        for s in range(8):
            vbase = (t * 8 + s) * 2
            for grp in range(8):
                e0 = (grp * 16) % 64
                col = vbase + (grp * 16) // 64
                work.append((t, s, grp, e0, col))
    for w0 in range(0, len(work), 32):
        wave = work[w0:w0 + 32]
        vecs = [
            plsc.load_gather(
                src, [iota + e0, jnp.full((16,), col, jnp.int32)])
            for (_, _, _, e0, col) in wave
        ]
        for (t, s, grp, _, _), vec in zip(wave, vecs):
            dst[t, s, pl.ds(grp * 16, 16)] = vec


@functools.partial(
    pl.kernel,
    out_type=jax.ShapeDtypeStruct((T5_BLKS, 8, 128), jnp.float32),
    mesh=_mesh,
    scratch_types=[
        pltpu.VMEM((64, 128), jnp.float32),
        pltpu.VMEM((64, 128), jnp.float32),
        pltpu.VMEM((8, 8, 128), jnp.float32),
        pltpu.VMEM((8, 8, 128), jnp.float32),
        pltpu.SemaphoreType.DMA,
        pltpu.SemaphoreType.DMA,
        pltpu.SemaphoreType.DMA,
        pltpu.SemaphoreType.DMA,
    ],
    compiler_params=pltpu.CompilerParams(use_tc_tiling_on_sc=True,
                                         needs_layout_passes=False),
)
def _sc_detile(wt, tail128, t5, in_a, in_b, out_a, out_b, ia, ib, oa, ob):
    wid = lax.axis_index("s") * NC + lax.axis_index("c")
    ins, outs, isems, osems = (in_a, in_b), (out_a, out_b), (ia, ib), (oa, ob)
    nfull = jnp.where(wid < 4, 245, 244)

    def fire_in(i, b):
        vb = wid + NW * i
        pltpu.async_copy(wt.at[:, pl.ds(vb * 128, 128)], ins[b], isems[b])

    def wait_in(b):
        pltpu.make_async_copy(wt.at[:, pl.ds(0, 128)], ins[b], isems[b]).wait()

    def wait_out(b):
        pltpu.make_async_copy(t5.at[pl.ds(0, 8)], outs[b], osems[b]).wait()

    for b in range(2):
        fire_in(b, b)

    def step(i):
        for b in range(2):
            @pl.when((i * 2 + b) < nfull)
            def _():
                j = i * 2 + b
                vb = wid + NW * j
                wait_in(b)
                @pl.when(j >= 2)
                def _():
                    wait_out(b)
                _transpose_block(ins[b], outs[b], 8)
                pltpu.async_copy(outs[b], t5.at[pl.ds(vb * 8, 8)], osems[b])
                @pl.when((j + 2) < nfull)
                def _():
                    fire_in(j + 2, b)

    pl.loop(0, 123)(step)
    for b in range(2):
        wait_out(b)

    @pl.when(wid == 4)
    def _():
        pltpu.sync_copy(tail128, in_a)
        _transpose_block(in_a, out_a, 4)
        pltpu.sync_copy(out_a.at[pl.ds(0, 4)],
                        t5.at[pl.ds(NFULL_VB * 8, 4)])


_scratch = (
    [pltpu.VMEM((NG, IDX_W), jnp.int32)]
    + [pltpu.VMEM((IDX_W, EMBED_DIM), jnp.float32) for _ in range(NBUF)]
    + [pltpu.SemaphoreType.DMA for _ in range(2 * NBUF)]
)


@functools.partial(
    pl.kernel,
    out_type=jax.ShapeDtypeStruct((B, EMBED_DIM), jnp.float32),
    mesh=_mesh,
    scratch_types=_scratch,
    compiler_params=pltpu.CompilerParams(use_tc_tiling_on_sc=False),
)
def _sc_gather(table, idx, out, idx_v, *bufs_and_sems):
    bufs = bufs_and_sems[:NBUF]
    gsems = bufs_and_sems[NBUF:2 * NBUF]
    ssems = bufs_and_sems[2 * NBUF:]
    wid = lax.axis_index("s") * NC + lax.axis_index("c")
    base = wid * B_PER_W
    pltpu.sync_copy(idx.at[wid], idx_v)

    def fire(g, b):
        pltpu.async_copy(table.at[idx_v.at[g]], bufs[b], gsems[b])

    def drain(b, sem):
        pltpu.make_async_copy(table.at[pl.ds(0, IDX_W)], bufs[b], sem).wait()

    for b in range(NBUF):
        fire(b, b)

    def step(i):
        for b in range(NBUF):
            g = i * NBUF + b
            drain(b, gsems[b])
            pltpu.async_copy(bufs[b], out.at[pl.ds(base + g * IDX_W, IDX_W)],
                             ssems[b])
        for b in range(NBUF):
            g_next = (i + 1) * NBUF + b

            @pl.when(g_next < NG)
            def _():
                drain(b, ssems[b])
                fire(g_next, b)

    pl.loop(0, NROUND)(step)
    for b in range(NBUF):
        drain(b, ssems[b])


def kernel(context, W):
    wt = W.T
    tail128 = jnp.pad(wt[:, NFULL_VB * 128:], ((0, 0), (0, 128 - TAIL_V)))
    t5 = _sc_detile(wt, tail128)
    table = t5.reshape(VOCAB, EMBED_DIM)
    idx = context.astype(jnp.int32).reshape(NW, NG, IDX_W)
    out = _sc_gather(table, idx)
    return out.reshape(BATCH, N_FIELDS, EMBED_DIM)


# speedup vs baseline: 1.4522x; 1.1348x over previous
import functools

import jax
import jax.numpy as jnp
from jax import lax
from jax.experimental import pallas as pl
from jax.experimental.pallas import tpu as pltpu
from jax.experimental.pallas import tpu_sc as plsc

VOCAB = 1000000
EMBED_DIM = 64
BATCH = 16384
N_FIELDS = 26

NC, NS = 2, 16
NW = NC * NS
B = BATCH * N_FIELDS
B_PER_W = B // NW
IDX_W = 128
NG = B_PER_W // IDX_W
NBUF = 8
NROUND = NG // NBUF

NFULL_VB = VOCAB // 128
TAIL_V = VOCAB - NFULL_VB * 128
T5_BLKS = VOCAB * EMBED_DIM // 1024

_mesh = plsc.VectorSubcoreMesh(core_axis_name="c", subcore_axis_name="s")

def _transpose_block(src, dst, nj):
    iota = lax.iota(jnp.int32, 16)
    scats = [iota * EMBED_DIM + r for r in range(8)]
    work = [(e, j) for j in range(nj) for e in range(64)]
    for w0 in range(0, len(work), 16):
        wave = work[w0:w0 + 16]
        vecs = [src[e, pl.ds(j * 16, 16)] for (e, j) in wave]
        for (e, j), vec in zip(wave, vecs):
            win = dst.at[pl.ds(j * 16 * EMBED_DIM + (e // 8) * 8,
                               15 * EMBED_DIM + 8)]
            plsc.store_scatter(win, [scats[e % 8]], vec)


@functools.partial(
    pl.kernel,
    out_type=jax.ShapeDtypeStruct((VOCAB * EMBED_DIM,), jnp.float32),
    mesh=_mesh,
    scratch_types=[
        pltpu.VMEM((64, 128), jnp.float32),
        pltpu.VMEM((64, 128), jnp.float32),
        pltpu.VMEM((128 * EMBED_DIM,), jnp.float32),
        pltpu.VMEM((128 * EMBED_DIM,), jnp.float32),
        pltpu.SemaphoreType.DMA,
        pltpu.SemaphoreType.DMA,
        pltpu.SemaphoreType.DMA,
        pltpu.SemaphoreType.DMA,
    ],
    compiler_params=pltpu.CompilerParams(use_tc_tiling_on_sc=True,
                                         needs_layout_passes=False),
)
def _sc_detile(wt, tail128, t5, in_a, in_b, out_a, out_b, ia, ib, oa, ob):
    wid = lax.axis_index("s") * NC + lax.axis_index("c")
    ins, outs, isems, osems = (in_a, in_b), (out_a, out_b), (ia, ib), (oa, ob)
    nfull = jnp.where(wid < 4, 245, 244)

    def fire_in(i, b):
        vb = wid + NW * i
        pltpu.async_copy(wt.at[:, pl.ds(vb * 128, 128)], ins[b], isems[b])

    def wait_in(b):
        pltpu.make_async_copy(wt.at[:, pl.ds(0, 128)], ins[b], isems[b]).wait()

    def wait_out(b):
        pltpu.make_async_copy(t5.at[pl.ds(0, 128 * EMBED_DIM)], outs[b],
                              osems[b]).wait()

    for b in range(2):
        fire_in(b, b)

    def step(i):
        for b in range(2):
            @pl.when((i * 2 + b) < nfull)
            def _():
                j = i * 2 + b
                vb = wid + NW * j
                wait_in(b)
                @pl.when(j >= 2)
                def _():
                    wait_out(b)
                _transpose_block(ins[b], outs[b], 8)
                pltpu.async_copy(outs[b],
                                 t5.at[pl.ds(vb * 128 * EMBED_DIM,
                                             128 * EMBED_DIM)],
                                 osems[b])
                @pl.when((j + 2) < nfull)
                def _():
                    fire_in(j + 2, b)

    pl.loop(0, 123)(step)
    for b in range(2):
        wait_out(b)

    @pl.when(wid == 4)
    def _():
        pltpu.sync_copy(tail128, in_a)
        _transpose_block(in_a, out_a, 4)
        pltpu.sync_copy(out_a.at[pl.ds(0, TAIL_V * EMBED_DIM)],
                        t5.at[pl.ds(NFULL_VB * 128 * EMBED_DIM,
                                    TAIL_V * EMBED_DIM)])


_scratch = (
    [pltpu.VMEM((NG, IDX_W), jnp.int32)]
    + [pltpu.VMEM((IDX_W, EMBED_DIM), jnp.float32) for _ in range(NBUF)]
    + [pltpu.SemaphoreType.DMA for _ in range(2 * NBUF)]
)


@functools.partial(
    pl.kernel,
    out_type=jax.ShapeDtypeStruct((B, EMBED_DIM), jnp.float32),
    mesh=_mesh,
    scratch_types=_scratch,
    compiler_params=pltpu.CompilerParams(use_tc_tiling_on_sc=False),
)
def _sc_gather(table, idx, out, idx_v, *bufs_and_sems):
    bufs = bufs_and_sems[:NBUF]
    gsems = bufs_and_sems[NBUF:2 * NBUF]
    ssems = bufs_and_sems[2 * NBUF:]
    wid = lax.axis_index("s") * NC + lax.axis_index("c")
    base = wid * B_PER_W
    pltpu.sync_copy(idx.at[wid], idx_v)

    def fire(g, b):
        pltpu.async_copy(table.at[idx_v.at[g]], bufs[b], gsems[b])

    def drain(b, sem):
        pltpu.make_async_copy(table.at[pl.ds(0, IDX_W)], bufs[b], sem).wait()

    for b in range(NBUF):
        fire(b, b)

    def step(i):
        for b in range(NBUF):
            g = i * NBUF + b
            drain(b, gsems[b])
            pltpu.async_copy(bufs[b], out.at[pl.ds(base + g * IDX_W, IDX_W)],
                             ssems[b])
        for b in range(NBUF):
            g_next = (i + 1) * NBUF + b

            @pl.when(g_next < NG)
            def _():
                drain(b, ssems[b])
                fire(g_next, b)

    pl.loop(0, NROUND)(step)
    for b in range(NBUF):
        drain(b, ssems[b])


def kernel(context, W):
    wt = W.T
    tail128 = jnp.pad(wt[:, NFULL_VB * 128:], ((0, 0), (0, 128 - TAIL_V)))
    t5 = _sc_detile(wt, tail128)
    table = t5.reshape(VOCAB, EMBED_DIM)
    idx = context.astype(jnp.int32).reshape(NW, NG, IDX_W)
    out = _sc_gather(table, idx)
    return out.reshape(BATCH, N_FIELDS, EMBED_DIM)


# speedup vs baseline: 1.9526x; 1.3446x over previous
import functools

import jax
import jax.numpy as jnp
from jax import lax
from jax.experimental import pallas as pl
from jax.experimental.pallas import tpu as pltpu
from jax.experimental.pallas import tpu_sc as plsc

VOCAB = 1000000
EMBED_DIM = 64
BATCH = 16384
N_FIELDS = 26

NC, NS = 2, 16
NW = NC * NS
B = BATCH * N_FIELDS
B_PER_W = B // NW
IDX_W = 128
NG = B_PER_W // IDX_W
NBUF = 8
NROUND = NG // NBUF

_mesh = plsc.VectorSubcoreMesh(core_axis_name="c", subcore_axis_name="s")

_scratch = (
    [pltpu.VMEM((NG, IDX_W), jnp.int32)]
    + [pltpu.VMEM((IDX_W, EMBED_DIM), jnp.float32) for _ in range(NBUF)]
    + [pltpu.SemaphoreType.DMA for _ in range(2 * NBUF)]
)


@functools.partial(
    pl.kernel,
    out_type=jax.ShapeDtypeStruct((B, EMBED_DIM), jnp.float32),
    mesh=_mesh,
    scratch_types=_scratch,
    compiler_params=pltpu.CompilerParams(use_tc_tiling_on_sc=False),
)
def _sc_gather(table, idx, out, idx_v, *bufs_and_sems):
    bufs = bufs_and_sems[:NBUF]
    gsems = bufs_and_sems[NBUF:2 * NBUF]
    ssems = bufs_and_sems[2 * NBUF:]
    wid = lax.axis_index("s") * NC + lax.axis_index("c")
    base = wid * B_PER_W
    pltpu.sync_copy(idx.at[wid], idx_v)

    def fire(g, b):
        pltpu.async_copy(table.at[idx_v.at[g]], bufs[b], gsems[b])

    def drain(b, sem):
        pltpu.make_async_copy(table.at[pl.ds(0, IDX_W)], bufs[b], sem).wait()

    for b in range(NBUF):
        fire(b, b)

    def step(i):
        for b in range(NBUF):
            g = i * NBUF + b
            drain(b, gsems[b])
            pltpu.async_copy(bufs[b], out.at[pl.ds(base + g * IDX_W, IDX_W)],
                             ssems[b])
        for b in range(NBUF):
            g_next = (i + 1) * NBUF + b

            @pl.when(g_next < NG)
            def _():
                drain(b, ssems[b])
                fire(g_next, b)

    pl.loop(0, NROUND)(step)
    for b in range(NBUF):
        drain(b, ssems[b])


def kernel(context, W):
    idx = context.astype(jnp.int32).reshape(NW, NG, IDX_W)
    out = _sc_gather(W, idx)
    return out.reshape(BATCH, N_FIELDS, EMBED_DIM)
